# baseline (speedup 1.0000x reference)
import functools

import jax
import jax.numpy as jnp
from jax import lax
from jax.experimental import pallas as pl
from jax.experimental.pallas import tpu as pltpu
from jax.experimental.pallas import tpu_sc as plsc

N_USERS = 20000
N_ITEMS = 30000
N = N_USERS + N_ITEMS
D = 64
DH = 32
HOPS = 3
E = 800000
L = 16

TILES = 16
SUB = 128
SUBS_PER_TILE = 392
EPAD = TILES * SUBS_PER_TILE * SUB
CH_SUB = 2
CHUNKS = SUBS_PER_TILE // CH_SUB

NBLK = 250
NBLKP = 256
NBLKS = N // NBLK
UBLKS = N_USERS // NBLK
TBLK_IT = 13
ZB = 50


def _body(xh0, cri, t2d,
          user_out, item_out, xn1, xn2,
          acc,
          ibuf0, ibuf1, ibuf2, ibuf3,
          rowsbuf_a, rowsbuf_b, accv, tv, zerov,
          sem_i0, sem_i1, sem_i2, sem_i3,
          sem_ga, sem_gb, sem_sa, sem_sb, sem_xn):
    c = lax.axis_index("c")
    s = lax.axis_index("s")

    isets = ((ibuf0, sem_i0), (ibuf1, sem_i1), (ibuf2, sem_i2),
             (ibuf3, sem_i3))
    rbufs = ((rowsbuf_a, sem_ga, sem_sa), (rowsbuf_b, sem_gb, sem_sb))

    zf = jnp.zeros((L,), jnp.float32)

    def _zrow(i, carry):
        zerov[i, pl.ds(0, L)] = zf
        zerov[i, pl.ds(L, L)] = zf
        return carry

    lax.fori_loop(0, ZB, _zrow, None)

    def _zero_acc(r0):
        for q in range(NBLK // ZB):
            pltpu.sync_copy(zerov, acc.at[pl.ds(r0 + q * ZB, ZB)])

    def _for_my_blocks(fn):
        def _blk(j, carry):
            g = s + j * TILES

            @pl.when(g < NBLKS)
            def _():
                fn(j, g)
            return carry

        lax.fori_loop(0, TBLK_IT, _blk, None)

    def _scale_block(stag, k):
        def _sgrp(gi, carry):
            r0 = gi * L
            vt = tv[pl.ds(r0, L)]
            f = vt
            if k > 0:
                d = 1.0 - vt
                for _ in range(k):
                    f = f * d
            for m in range(L):
                f_m = f[m]
                stag[r0 + m, pl.ds(0, L)] = accv[r0 + m, pl.ds(0, L)] * f_m
                stag[r0 + m, pl.ds(L, L)] = accv[r0 + m, pl.ds(L, L)] * f_m
            return carry

        lax.fori_loop(0, NBLKP // L, _sgrp, None)

    def _write_out(stag, g, hop):
        src = stag.at[pl.ds(0, NBLK)]

        @pl.when(g < UBLKS)
        def _():
            pltpu.sync_copy(
                src, user_out.at[pl.ds(g * NBLK, NBLK), hop, pl.ds(c * DH, DH)])

        @pl.when(g >= UBLKS)
        def _():
            pltpu.sync_copy(
                src,
                item_out.at[pl.ds(g * NBLK - N_USERS, NBLK), hop,
                            pl.ds(c * DH, DH)])

    def _block_pass(k, kind, rd_src, nxt):
        do_xn = kind == 1 and k < HOPS

        def _xn_wait_zero(g_prev):
            pltpu.make_async_copy(accv.at[pl.ds(0, NBLK)],
                                  nxt.at[pl.ds(g_prev * NBLK, NBLK)],
                                  sem_xn).wait()
            _zero_acc(g_prev * NBLK)

        def _fn(j, g):
            stag = rowsbuf_a
            pltpu.sync_copy(rd_src.at[pl.ds(g * NBLK, NBLK)],
                            accv.at[pl.ds(0, NBLK)])
            if do_xn:
                @pl.when(j >= 1)
                def _():
                    _xn_wait_zero(g - TILES)
                pltpu.async_copy(accv.at[pl.ds(0, NBLK)],
                                 nxt.at[pl.ds(g * NBLK, NBLK)], sem_xn)
            if kind == 0:
                _zero_acc(g * NBLK)
            pltpu.sync_copy(t2d.at[g], tv.at[pl.ds(0, NBLK)])
            _scale_block(stag, k)
            _write_out(stag, g, k)

        _for_my_blocks(_fn)
        if do_xn:
            last_g1 = s + (TBLK_IT - 1) * TILES
            last_g2 = s + (TBLK_IT - 2) * TILES

            @pl.when(last_g1 < NBLKS)
            def _():
                _xn_wait_zero(last_g1)

            @pl.when(last_g1 >= NBLKS)
            def _():
                _xn_wait_zero(last_g2)

    _block_pass(0, 0, xh0.at[c], None)
    plsc.subcore_barrier()

    srcs = [xh0.at[c], xn1.at[c], xn2.at[c]]
    nxts = [None, xn1.at[c], xn2.at[c]]
    for k in range(1, HOPS + 1):
        src = srcs[k - 1]
        base = s * SUBS_PER_TILE

        def _issue_idx(ch, q, base=base):
            ibuf, sem_i = isets[q]
            sb = base + ch * CH_SUB
            return pltpu.async_copy(cri.at[pl.ds(sb, CH_SUB)], ibuf, sem_i)

        def _wait_idx(q):
            ibuf, sem_i = isets[q]
            pltpu.make_async_copy(cri.at[pl.ds(0, CH_SUB)], ibuf, sem_i).wait()

        def _issue_gather(q, p, src=src):
            ibuf = isets[q][0]
            rowsbuf, sem_g, _ = rbufs[p]
            for j in range(CH_SUB):
                pltpu.async_copy(src.at[ibuf.at[j, 0]],
                                 rowsbuf.at[pl.ds(j * SUB, SUB)], sem_g)

        def _wait_gather(q, p, src=src):
            ibuf = isets[q][0]
            rowsbuf, sem_g, _ = rbufs[p]
            for j in range(CH_SUB):
                pltpu.make_async_copy(src.at[ibuf.at[j, 0]],
                                      rowsbuf.at[pl.ds(j * SUB, SUB)],
                                      sem_g).wait()

        def _scale(q, p):
            ibuf = isets[q][0]
            rowsbuf = rbufs[p][0]
            for j in range(CH_SUB):
                def _sgrp(gi, carry2, j=j):
                    e0 = gi * L
                    vv = lax.bitcast_convert_type(ibuf[j, 2, pl.ds(e0, L)],
                                                  jnp.float32)
                    for m in range(L):
                        v = vv[m]
                        r = j * SUB + e0 + m
                        rowsbuf[r, pl.ds(0, L)] = rowsbuf[r, pl.ds(0, L)] * v
                        rowsbuf[r, pl.ds(L, L)] = rowsbuf[r, pl.ds(L, L)] * v
                    return carry2

                lax.fori_loop(0, SUB // L, _sgrp, None)

        def _issue_scatter(q, p):
            ibuf = isets[q][0]
            rowsbuf, _, sem_s = rbufs[p]
            for j in range(CH_SUB):
                pltpu.async_copy(rowsbuf.at[pl.ds(j * SUB, SUB)],
                                 acc.at[ibuf.at[j, 1]], sem_s, add=True)

        def _wait_scatter(q, p):
            ibuf = isets[q][0]
            rowsbuf, _, sem_s = rbufs[p]
            for j in range(CH_SUB):
                pltpu.make_async_copy(rowsbuf.at[pl.ds(j * SUB, SUB)],
                                      acc.at[ibuf.at[j, 1]], sem_s).wait()

        def _gss(q, p, src=src):
            ibuf = isets[q][0]
            rowsbuf, sem_g, sem_s = rbufs[p]
            for j in range(CH_SUB):
                pltpu.make_async_copy(src.at[ibuf.at[j, 0]],
                                      rowsbuf.at[pl.ds(j * SUB, SUB)],
                                      sem_g).wait()

                def _sgrp(gi, carry2, j=j):
                    e0 = gi * L
                    vv = lax.bitcast_convert_type(ibuf[j, 2, pl.ds(e0, L)],
                                                  jnp.float32)
                    for m in range(L):
                        v = vv[m]
                        r = j * SUB + e0 + m
                        rowsbuf[r, pl.ds(0, L)] = rowsbuf[r, pl.ds(0, L)] * v
                        rowsbuf[r, pl.ds(L, L)] = rowsbuf[r, pl.ds(L, L)] * v
                    return carry2

                lax.fori_loop(0, SUB // L, _sgrp, None, unroll=2)
                pltpu.async_copy(rowsbuf.at[pl.ds(j * SUB, SUB)],
                                 acc.at[ibuf.at[j, 1]], sem_s, add=True)

        def _phase(ch, u):
            q2, q1, q0 = (u + 2) % 4, (u + 1) % 4, u % 4
            p1, p0 = (u + 1) % 2, u % 2

            @pl.when(ch + 2 < CHUNKS)
            def _():
                _issue_idx(ch + 2, q2)

            @pl.when(ch + 1 < CHUNKS)
            def _():
                _wait_idx(q1)

                @pl.when(ch >= 1)
                def _():
                    _wait_scatter((u + 3) % 4, p1)
                _issue_gather(q1, p1)

            _gss(q0, p0)

        _issue_idx(1, 1)
        _issue_idx(0, 0).wait()
        _issue_gather(0, 0)

        def _pipe(ch4, carry):
            for u in range(4):
                _phase(ch4 * 4 + u, u)
            return carry

        lax.fori_loop(0, CHUNKS // 4, _pipe, None)
        _wait_scatter(2, 0)
        _wait_scatter(3, 1)

        plsc.subcore_barrier()
        _block_pass(k, 1, acc, nxts[k] if k < HOPS else None)
        plsc.subcore_barrier()


_propagate = functools.partial(
    pl.kernel,
    out_type=(
        jax.ShapeDtypeStruct((N_USERS, HOPS + 1, D), jnp.float32),
        jax.ShapeDtypeStruct((N_ITEMS, HOPS + 1, D), jnp.float32),
        jax.ShapeDtypeStruct((2, N, DH), jnp.float32),
        jax.ShapeDtypeStruct((2, N, DH), jnp.float32),
    ),
    mesh=plsc.VectorSubcoreMesh(core_axis_name="c", subcore_axis_name="s"),
    compiler_params=pltpu.CompilerParams(use_tc_tiling_on_sc=False),
    scratch_types=[
        pltpu.VMEM_SHARED((N, DH), jnp.float32),
        pltpu.VMEM((CH_SUB, 3, SUB), jnp.int32),
        pltpu.VMEM((CH_SUB, 3, SUB), jnp.int32),
        pltpu.VMEM((CH_SUB, 3, SUB), jnp.int32),
        pltpu.VMEM((CH_SUB, 3, SUB), jnp.int32),
        pltpu.VMEM((CH_SUB * SUB, DH), jnp.float32),
        pltpu.VMEM((CH_SUB * SUB, DH), jnp.float32),
        pltpu.VMEM((NBLKP, DH), jnp.float32),
        pltpu.VMEM((NBLKP,), jnp.float32),
        pltpu.VMEM((ZB, DH), jnp.float32),
    ] + [pltpu.SemaphoreType.DMA] * 9,
)(_body)


def kernel(user_embed, item_embed, row, col, vals, user_t, item_t):
    all_embed = jnp.concatenate([user_embed, item_embed], axis=0)
    xh0 = jnp.stack([all_embed[:, :DH], all_embed[:, DH:]])
    t2d = jnp.concatenate([user_t, item_t], axis=0)[:, 0].reshape(NBLKS, NBLK)
    pad = EPAD - E

    def p2d(a, fill):
        return jnp.concatenate([a, jnp.full((pad,), fill, a.dtype)]).reshape(
            -1, SUB)

    colp = p2d(col, 0)
    rowp = p2d(row, 0)
    valsp = lax.bitcast_convert_type(p2d(vals, jnp.float32(0.0)), jnp.int32)
    cri = jnp.stack([colp, rowp, valsp], axis=1)
    user_out, item_out, _, _ = _propagate(xh0, cri, t2d)
    return user_out, item_out


# speedup vs baseline: 1.9714x; 1.9714x over previous
import functools

import jax
import jax.numpy as jnp
from jax import lax
from jax.experimental import pallas as pl
from jax.experimental.pallas import tpu as pltpu
from jax.experimental.pallas import tpu_sc as plsc

N_USERS = 20000
N_ITEMS = 30000
N = N_USERS + N_ITEMS
D = 64
DH = 32
HOPS = 3
E = 800000
L = 16

TILES = 16
SUB = 128
SUBS_PER_TILE = 392
EPAD = TILES * SUBS_PER_TILE * SUB
CH_SUB = 2
CHUNKS = SUBS_PER_TILE // CH_SUB

NBLK = 250
NBLKP = 256
NBLKS = N // NBLK
UBLKS = N_USERS // NBLK
TBLK_IT = 13
ZB = 50


def _body(xh0, cri, t2d,
          user_out, item_out, xn1, xn2,
          acc,
          ibuf0, ibuf1, ibuf2, ibuf3,
          rowsbuf_a, rowsbuf_b, accv, tv, zerov,
          sem_i0, sem_i1, sem_i2, sem_i3,
          sem_ga, sem_gb, sem_sa, sem_sb, sem_xn):
    c = lax.axis_index("c")
    s = lax.axis_index("s")

    isets = ((ibuf0, sem_i0), (ibuf1, sem_i1), (ibuf2, sem_i2),
             (ibuf3, sem_i3))
    rbufs = ((rowsbuf_a, sem_ga, sem_sa), (rowsbuf_b, sem_gb, sem_sb))

    zf = jnp.zeros((L,), jnp.float32)

    def _zrow(i, carry):
        zerov[i, pl.ds(0, L)] = zf
        zerov[i, pl.ds(L, L)] = zf
        return carry

    lax.fori_loop(0, ZB, _zrow, None)

    def _zero_acc(r0):
        for q in range(NBLK // ZB):
            pltpu.sync_copy(zerov, acc.at[pl.ds(r0 + q * ZB, ZB)])

    def _for_my_blocks(fn):
        def _blk(j, carry):
            g = s + j * TILES

            @pl.when(g < NBLKS)
            def _():
                fn(j, g)
            return carry

        lax.fori_loop(0, TBLK_IT, _blk, None)

    def _scale_block(stag, k):
        def _sgrp(gi, carry):
            r0 = gi * L
            vt = tv[pl.ds(r0, L)]
            f = vt
            if k > 0:
                d = 1.0 - vt
                for _ in range(k):
                    f = f * d
            for m in range(L):
                f_m = f[m]
                stag[r0 + m, pl.ds(0, L)] = accv[r0 + m, pl.ds(0, L)] * f_m
                stag[r0 + m, pl.ds(L, L)] = accv[r0 + m, pl.ds(L, L)] * f_m
            return carry

        lax.fori_loop(0, NBLKP // L, _sgrp, None)

    def _write_out(stag, g, hop):
        src = stag.at[pl.ds(0, NBLK)]

        @pl.when(g < UBLKS)
        def _():
            pltpu.sync_copy(
                src, user_out.at[pl.ds(g * NBLK, NBLK), hop, pl.ds(c * DH, DH)])

        @pl.when(g >= UBLKS)
        def _():
            pltpu.sync_copy(
                src,
                item_out.at[pl.ds(g * NBLK - N_USERS, NBLK), hop,
                            pl.ds(c * DH, DH)])

    def _block_pass(k, kind, rd_src, nxt):
        do_xn = kind == 1 and k < HOPS

        def _xn_wait_zero(g_prev):
            pltpu.make_async_copy(accv.at[pl.ds(0, NBLK)],
                                  nxt.at[pl.ds(g_prev * NBLK, NBLK)],
                                  sem_xn).wait()
            _zero_acc(g_prev * NBLK)

        def _fn(j, g):
            stag = rowsbuf_a
            pltpu.sync_copy(rd_src.at[pl.ds(g * NBLK, NBLK)],
                            accv.at[pl.ds(0, NBLK)])
            if do_xn:
                @pl.when(j >= 1)
                def _():
                    _xn_wait_zero(g - TILES)
                pltpu.async_copy(accv.at[pl.ds(0, NBLK)],
                                 nxt.at[pl.ds(g * NBLK, NBLK)], sem_xn)
            if kind == 0:
                _zero_acc(g * NBLK)
            pltpu.sync_copy(t2d.at[g], tv.at[pl.ds(0, NBLK)])
            _scale_block(stag, k)
            _write_out(stag, g, k)

        _for_my_blocks(_fn)
        if do_xn:
            last_g1 = s + (TBLK_IT - 1) * TILES
            last_g2 = s + (TBLK_IT - 2) * TILES

            @pl.when(last_g1 < NBLKS)
            def _():
                _xn_wait_zero(last_g1)

            @pl.when(last_g1 >= NBLKS)
            def _():
                _xn_wait_zero(last_g2)

    _block_pass(0, 0, xh0.at[c], None)
    plsc.subcore_barrier()

    srcs = [xh0.at[c], xn1.at[c], xn2.at[c]]
    nxts = [None, xn1.at[c], xn2.at[c]]
    for k in range(1, HOPS + 1):
        src = srcs[k - 1]
        base = s * SUBS_PER_TILE

        def _issue_idx(ch, q, base=base):
            ibuf, sem_i = isets[q]
            sb = base + ch * CH_SUB
            return pltpu.async_copy(cri.at[pl.ds(sb, CH_SUB)], ibuf, sem_i)

        def _wait_idx(q):
            ibuf, sem_i = isets[q]
            pltpu.make_async_copy(cri.at[pl.ds(0, CH_SUB)], ibuf, sem_i).wait()

        def _issue_gather(q, p, src=src):
            ibuf = isets[q][0]
            rowsbuf, sem_g, _ = rbufs[p]
            for j in range(CH_SUB):
                pltpu.async_copy(src.at[ibuf.at[j, 0]],
                                 rowsbuf.at[pl.ds(j * SUB, SUB)], sem_g)

        def _wait_gather(q, p, src=src):
            ibuf = isets[q][0]
            rowsbuf, sem_g, _ = rbufs[p]
            for j in range(CH_SUB):
                pltpu.make_async_copy(src.at[ibuf.at[j, 0]],
                                      rowsbuf.at[pl.ds(j * SUB, SUB)],
                                      sem_g).wait()

        def _scale(q, p):
            ibuf = isets[q][0]
            rowsbuf = rbufs[p][0]
            for j in range(CH_SUB):
                def _sgrp(gi, carry2, j=j):
                    e0 = gi * L
                    vv = lax.bitcast_convert_type(ibuf[j, 2, pl.ds(e0, L)],
                                                  jnp.float32)
                    for m in range(L):
                        v = vv[m]
                        r = j * SUB + e0 + m
                        rowsbuf[r, pl.ds(0, L)] = rowsbuf[r, pl.ds(0, L)] * v
                        rowsbuf[r, pl.ds(L, L)] = rowsbuf[r, pl.ds(L, L)] * v
                    return carry2

                lax.fori_loop(0, SUB // L, _sgrp, None)

        def _issue_scatter(q, p):
            ibuf = isets[q][0]
            rowsbuf, _, sem_s = rbufs[p]
            for j in range(CH_SUB):
                pltpu.async_copy(rowsbuf.at[pl.ds(j * SUB, SUB)],
                                 acc.at[ibuf.at[j, 1]], sem_s, add=True)

        def _wait_scatter(q, p):
            ibuf = isets[q][0]
            rowsbuf, _, sem_s = rbufs[p]
            for j in range(CH_SUB):
                pltpu.make_async_copy(rowsbuf.at[pl.ds(j * SUB, SUB)],
                                      acc.at[ibuf.at[j, 1]], sem_s).wait()

        def _gss(q, p, src=src):
            ibuf = isets[q][0]
            rowsbuf, sem_g, sem_s = rbufs[p]
            for j in range(CH_SUB):
                pltpu.make_async_copy(src.at[ibuf.at[j, 0]],
                                      rowsbuf.at[pl.ds(j * SUB, SUB)],
                                      sem_g).wait()

                def _sgrp(gi, carry2, j=j):
                    e0 = gi * L
                    vv = lax.bitcast_convert_type(ibuf[j, 2, pl.ds(e0, L)],
                                                  jnp.float32)
                    for m in range(L):
                        v = vv[m]
                        r = j * SUB + e0 + m
                        rowsbuf[r, pl.ds(0, L)] = rowsbuf[r, pl.ds(0, L)] * v
                        rowsbuf[r, pl.ds(L, L)] = rowsbuf[r, pl.ds(L, L)] * v
                    return carry2

                lax.fori_loop(0, SUB // L, _sgrp, None)
                pltpu.async_copy(rowsbuf.at[pl.ds(j * SUB, SUB)],
                                 acc.at[ibuf.at[j, 1]], sem_s, add=True)

        def _phase(ch, u):
            q2, q1, q0 = (u + 2) % 4, (u + 1) % 4, u % 4
            p1, p0 = (u + 1) % 2, u % 2

            @pl.when(ch + 2 < CHUNKS)
            def _():
                _issue_idx(ch + 2, q2)

            @pl.when(ch + 1 < CHUNKS)
            def _():
                _wait_idx(q1)

                @pl.when(ch >= 1)
                def _():
                    _wait_scatter((u + 3) % 4, p1)
                _issue_gather(q1, p1)

            _gss(q0, p0)

        _issue_idx(1, 1)
        _issue_idx(0, 0).wait()
        _issue_gather(0, 0)

        def _pipe(ch4, carry):
            for u in range(4):
                _phase(ch4 * 4 + u, u)
            return carry

        lax.fori_loop(0, CHUNKS // 4, _pipe, None)
        _wait_scatter(2, 0)
        _wait_scatter(3, 1)

        plsc.subcore_barrier()
        _block_pass(k, 1, acc, nxts[k] if k < HOPS else None)
        plsc.subcore_barrier()


_propagate = functools.partial(
    pl.kernel,
    out_type=(
        jax.ShapeDtypeStruct((N_USERS, HOPS + 1, D), jnp.float32),
        jax.ShapeDtypeStruct((N_ITEMS, HOPS + 1, D), jnp.float32),
        jax.ShapeDtypeStruct((2, N, DH), jnp.float32),
        jax.ShapeDtypeStruct((2, N, DH), jnp.float32),
    ),
    mesh=plsc.VectorSubcoreMesh(core_axis_name="c", subcore_axis_name="s"),
    compiler_params=pltpu.CompilerParams(use_tc_tiling_on_sc=False),
    scratch_types=[
        pltpu.VMEM_SHARED((N, DH), jnp.float32),
        pltpu.VMEM((CH_SUB, 3, SUB), jnp.int32),
        pltpu.VMEM((CH_SUB, 3, SUB), jnp.int32),
        pltpu.VMEM((CH_SUB, 3, SUB), jnp.int32),
        pltpu.VMEM((CH_SUB, 3, SUB), jnp.int32),
        pltpu.VMEM((CH_SUB * SUB, DH), jnp.float32),
        pltpu.VMEM((CH_SUB * SUB, DH), jnp.float32),
        pltpu.VMEM((NBLKP, DH), jnp.float32),
        pltpu.VMEM((NBLKP,), jnp.float32),
        pltpu.VMEM((ZB, DH), jnp.float32),
    ] + [pltpu.SemaphoreType.DMA] * 9,
)(_body)


def kernel(user_embed, item_embed, row, col, vals, user_t, item_t):
    all_embed = jnp.concatenate([user_embed, item_embed], axis=0)
    xh0 = jnp.stack([all_embed[:, :DH], all_embed[:, DH:]])
    t2d = jnp.concatenate([user_t, item_t], axis=0)[:, 0].reshape(NBLKS, NBLK)
    pad = EPAD - E

    def p2d(a, fill):
        return jnp.concatenate([a, jnp.full((pad,), fill, a.dtype)]).reshape(
            -1, SUB)

    colp = p2d(col, 0)
    rowp = p2d(row, 0)
    valsp = lax.bitcast_convert_type(p2d(vals, jnp.float32(0.0)), jnp.int32)
    cri = jnp.stack([colp, rowp, valsp], axis=1)
    user_out, item_out, _, _ = _propagate(xh0, cri, t2d)
    return user_out, item_out
